# baseline (speedup 1.0000x reference)
import jax
import jax.numpy as jnp
from jax.experimental import pallas as pl
from jax.experimental.pallas import tpu as pltpu

BM = 512


def _fc_kernel(seq_ref, w_ref, seq_raw_ref):
    seq_raw_ref[...] = jnp.dot(
        seq_ref[...], w_ref[...].T, preferred_element_type=jnp.float32
    ).astype(jnp.bfloat16)


def _agg_kernel(seq_raw_ref, adj_ref, out_ref):
    acc = jnp.dot(adj_ref[...].astype(jnp.bfloat16), seq_raw_ref[...],
                  preferred_element_type=jnp.float32)
    out_ref[...] = jnp.maximum(acc, 0.0)


@jax.jit
def kernel(seq, adj, W):
    n, d_in = seq.shape
    d_out = W.shape[0]
    seq_raw = pl.pallas_call(
        _fc_kernel,
        out_shape=jax.ShapeDtypeStruct((n, d_out), jnp.bfloat16),
    )(seq, W)
    return pl.pallas_call(
        _agg_kernel,
        grid=(pl.cdiv(n, BM),),
        in_specs=[
            pl.BlockSpec((n, d_out), lambda i: (0, 0)),
            pl.BlockSpec((BM, n), lambda i: (i, 0)),
        ],
        out_specs=pl.BlockSpec((BM, d_out), lambda i: (i, 0)),
        out_shape=jax.ShapeDtypeStruct((n, d_out), jnp.float32),
        compiler_params=pltpu.CompilerParams(
            dimension_semantics=("parallel",),
        ),
    )(seq_raw, adj)


# speedup vs baseline: 1.0321x; 1.0321x over previous
import jax
import jax.numpy as jnp
from jax.experimental import pallas as pl
from jax.experimental.pallas import tpu as pltpu

BM = 512
HALF = BM // 2


def _gcn_kernel(seq_ref, w_ref, adj_a_ref, adj_b_ref, out_ref, seq_raw_ref):
    @pl.when(pl.program_id(0) == 0)
    def _():
        seq_raw_ref[...] = jnp.dot(
            seq_ref[...], w_ref[...].T, preferred_element_type=jnp.float32
        ).astype(jnp.bfloat16)

    s = seq_raw_ref[...]
    out_ref[:HALF, :] = jnp.maximum(
        jnp.dot(adj_a_ref[...].astype(jnp.bfloat16), s,
                preferred_element_type=jnp.float32), 0.0)
    out_ref[HALF:, :] = jnp.maximum(
        jnp.dot(adj_b_ref[...].astype(jnp.bfloat16), s,
                preferred_element_type=jnp.float32), 0.0)


@jax.jit
def kernel(seq, adj, W):
    n, d_in = seq.shape
    d_out = W.shape[0]
    return pl.pallas_call(
        _gcn_kernel,
        grid=(pl.cdiv(n, BM),),
        in_specs=[
            pl.BlockSpec((n, d_in), lambda i: (0, 0)),
            pl.BlockSpec((d_out, d_in), lambda i: (0, 0)),
            pl.BlockSpec((HALF, n), lambda i: (2 * i, 0)),
            pl.BlockSpec((HALF, n), lambda i: (2 * i + 1, 0)),
        ],
        out_specs=pl.BlockSpec((BM, d_out), lambda i: (i, 0)),
        out_shape=jax.ShapeDtypeStruct((n, d_out), jnp.float32),
        scratch_shapes=[pltpu.VMEM((n, d_out), jnp.bfloat16)],
    )(seq, W, adj, adj)


# speedup vs baseline: 1.0461x; 1.0136x over previous
import jax
import jax.numpy as jnp
from jax.experimental import pallas as pl
from jax.experimental.pallas import tpu as pltpu

BM = 256


def _gcn_kernel(seq_ref, w_ref, adj_ref, out_ref, seq_raw_ref):
    @pl.when(pl.program_id(0) == 0)
    def _():
        seq_raw_ref[...] = jnp.dot(
            seq_ref[...], w_ref[...].T, preferred_element_type=jnp.float32
        ).astype(jnp.bfloat16)

    acc = jnp.dot(adj_ref[...].astype(jnp.bfloat16), seq_raw_ref[...],
                  preferred_element_type=jnp.float32)
    out_ref[...] = jnp.maximum(acc, 0.0)


@jax.jit
def kernel(seq, adj, W):
    n, d_in = seq.shape
    d_out = W.shape[0]
    return pl.pallas_call(
        _gcn_kernel,
        grid=(pl.cdiv(n, BM),),
        in_specs=[
            pl.BlockSpec((n, d_in), lambda i: (0, 0)),
            pl.BlockSpec((d_out, d_in), lambda i: (0, 0)),
            pl.BlockSpec((BM, n), lambda i: (i, 0)),
        ],
        out_specs=pl.BlockSpec((BM, d_out), lambda i: (i, 0)),
        out_shape=jax.ShapeDtypeStruct((n, d_out), jnp.float32),
        scratch_shapes=[pltpu.VMEM((n, d_out), jnp.bfloat16)],
    )(seq, W, adj)
